# baseline (speedup 1.0000x reference)
import functools

import jax
import jax.numpy as jnp
from jax import lax
from jax.experimental import pallas as pl
from jax.experimental.pallas import tpu as pltpu
from jax.experimental.pallas import tpu_sc as plsc

D_OUT = 8
NC, NS = 2, 16
NW = NC * NS
TILE_L, TILE_B = 8, 128
SLAB = TILE_L * TILE_B
EX_CHK = 4000


def _prep_body(wt_ref, st_ref, qt_ref, ct_ref, mt_ref,
               ww_ref, sw_ref, qw_ref, cw_ref, mw_ref,
               wb_ref, sb_ref, qb_ref, cb_ref, mb_ref,
               wt8_ref, qt8_ref, cms_ref):
    f32 = jnp.float32
    wt8_ref[...] = jnp.dot(wt_ref[...], ww_ref[...].T, preferred_element_type=f32)
    qt8_ref[...] = jnp.dot(qt_ref[...], qw_ref[...][:, :2].T, preferred_element_type=f32)
    st8 = jnp.dot(st_ref[...], sw_ref[...].T, preferred_element_type=f32)
    ct8 = jnp.dot(ct_ref[...], cw_ref[...].T, preferred_element_type=f32)
    mt8 = jnp.dot(mt_ref[...], mw_ref[...].T, preferred_element_type=f32)
    bias = wb_ref[...] + sb_ref[...] + qb_ref[...] + cb_ref[...] + mb_ref[...]
    cm = (ct8[:, None, :] + mt8[None, :, :]).reshape(8, D_OUT)
    cms = st8[:, None, :] + cm[None, :, :] + bias[None, None, :]
    cms_ref[...] = cms


def _transpose_body(src_ref, dst_ref):
    t = src_ref[...].T
    dst_ref[...] = jnp.concatenate(
        [t, jnp.zeros((t.shape[0], 1), jnp.float32)], axis=1)


def _sc_body(ex_t, wt8, qt8, cms, g_ex, g_w, g_q, g_s, g_c, g_m, out,
             idx, rows, acc, twt, tqt, tcms, stage, sem_i, sem_g, sem_o, sem_t,
             *, n_lb):
    w = lax.axis_index("s") * NC + lax.axis_index("c")
    lane = lax.iota(jnp.int32, 16)

    p9 = (lane >> 3) * 9 + (lane & 7)
    for src, dst in ((wt8, twt), (qt8, tqt), (cms, tcms)):
        cp = pltpu.make_async_copy(src, stage.at[pl.ds(0, src.shape[0])], sem_t)
        cp.start()
        cp.wait()

        @plsc.parallel_loop(0, src.shape[0] // 16, unroll=4)
        def _expand(k, dst=dst):
            plsc.store_scatter(dst, [p9 + k * 18], stage[pl.ds(k * 16, 16)])

    def load_fold_gather(lb, s):
        gi = idx[s]
        cps = [pltpu.make_async_copy(g.at[lb, w], gi[i], sem_i[s])
               for i, g in enumerate((g_ex, g_w, g_q, g_s, g_c, g_m))]
        for cp in cps:
            cp.start()
        with jax.named_scope("idx_wait"):
            for cp in cps:
                cp.wait()

        @plsc.parallel_loop(0, SLAB // 16, unroll=4)
        def _fold(k):
            ks = pl.ds(k * 16, 16)
            gi[1][ks] = gi[1][ks] * 9
            gi[2][ks] = gi[2][ks] * 9
            gi[3][ks] = gi[3][ks] * 72 + gi[4][ks] * 36 + gi[5][ks] * 9

        pltpu.make_async_copy(ex_t.at[gi[0]], rows[s], sem_g[s]).start()

    def do_round(r, s):
        @pl.when(r + 1 < n_lb)
        def _prefetch():
            load_fold_gather(r + 1, 1 - s)

        with jax.named_scope("gather_wait"):
            pltpu.make_async_copy(ex_t.at[idx[s][0]], rows[s], sem_g[s]).wait()

        @pl.when(r >= 2)
        def _drain_out():
            pltpu.make_async_copy(acc[s], out.at[pl.ds((r - 2) * TILE_L, TILE_L), w],
                                  sem_o[s]).wait()

        gi, rex, a = idx[s], rows[s], acc[s]
        iw, iq, icms = gi[1], gi[2], gi[3]

        zero16 = jnp.broadcast_to(jnp.int32(0), (16,))

        sc_add = jax.named_scope("adds")
        sc_add.__enter__()

        @plsc.parallel_loop(0, SLAB // 16, unroll=2)
        def _add(j):
            js = pl.ds(j * 16, 16)
            tok = j * 16 + lane
            w8 = iw[js]
            q8 = iq[js]
            c8 = icms[js]
            li = j >> 3
            base = (j & 7) * 16
            for d in range(D_OUT):
                v = (plsc.load_gather(rex, [tok, zero16 + d])
                     + plsc.load_gather(twt, [w8])
                     + plsc.load_gather(tqt, [q8])
                     + plsc.load_gather(tcms, [c8]))
                a[li, pl.ds(d * TILE_B + base, 16)] = v
                if d < D_OUT - 1:
                    w8 = w8 + 1
                    q8 = q8 + 1
                    c8 = c8 + 1

        sc_add.__exit__(None, None, None)
        pltpu.make_async_copy(a, out.at[pl.ds(r * TILE_L, TILE_L), w],
                              sem_o[s]).start()

    load_fold_gather(0, 0)

    @pl.loop(0, (n_lb + 1) // 2)
    def _round_pair(h):
        for sub in (0, 1):
            r = h * 2 + sub

            @pl.when(r < n_lb)
            def _do(r=r, sub=sub):
                do_round(r, sub)

    for s, r in ((n_lb % 2, n_lb - 2), ((n_lb - 1) % 2, n_lb - 1)):
        pltpu.make_async_copy(acc[s], out.at[pl.ds(r * TILE_L, TILE_L), w],
                              sem_o[s]).wait()


def kernel(exercise_id, weight_id, exercise_sequence, equipment_id, core, metric_type,
           exercise_table, weight_table, seq_table, equipment_table, core_table, metric_table,
           weight_fc_w, weight_fc_b, seq_fc_w, seq_fc_b, equipment_fc_w, equipment_fc_b,
           core_fc_w, core_fc_b, metric_fc_w, metric_fc_b):
    B, L = exercise_id.shape
    V = exercise_table.shape[0]
    f32 = jnp.float32
    n_lb, n_bb = L // TILE_L, B // TILE_B

    wt8, qt8, cms3 = pl.pallas_call(
        _prep_body,
        out_shape=[
            jax.ShapeDtypeStruct(weight_table.shape[:1] + (D_OUT,), f32),
            jax.ShapeDtypeStruct(equipment_table.shape[:1] + (D_OUT,), f32),
            jax.ShapeDtypeStruct((seq_table.shape[0], 8, D_OUT), f32),
        ],
    )(weight_table, seq_table, equipment_table, core_table, metric_table,
      weight_fc_w, seq_fc_w, equipment_fc_w, core_fc_w, metric_fc_w,
      weight_fc_b, seq_fc_b, equipment_fc_b, core_fc_b, metric_fc_b)

    ex_lin = pl.pallas_call(
        _transpose_body,
        out_shape=jax.ShapeDtypeStruct((V, 9), f32),
    )(exercise_table.T)

    def tiled_view(a):
        return (a.T.reshape(n_lb, TILE_L, n_bb, TILE_B)
                .transpose(0, 2, 1, 3).reshape(n_lb, n_bb, SLAB))

    mesh = plsc.VectorSubcoreMesh(core_axis_name="c", subcore_axis_name="s",
                                  num_cores=NC, num_subcores=NS)
    sc = pl.kernel(
        functools.partial(_sc_body, n_lb=n_lb),
        out_type=jax.ShapeDtypeStruct((L, n_bb, SLAB), f32),
        mesh=mesh,
        scratch_types=[
            [[pltpu.VMEM((SLAB,), jnp.int32) for _ in range(6)] for _ in range(2)],
            [pltpu.VMEM((SLAB, 9), f32) for _ in range(2)],
            [pltpu.VMEM((TILE_L, SLAB), f32) for _ in range(2)],
            pltpu.VMEM((weight_table.shape[0] * 9,), f32),
            pltpu.VMEM((equipment_table.shape[0] * 9,), f32),
            pltpu.VMEM((seq_table.shape[0] * 8 * 9,), f32),
            pltpu.VMEM((seq_table.shape[0] * 8 * D_OUT,), f32),
            [pltpu.SemaphoreType.DMA for _ in range(2)],
            [pltpu.SemaphoreType.DMA for _ in range(2)],
            [pltpu.SemaphoreType.DMA for _ in range(2)],
            pltpu.SemaphoreType.DMA,
        ],
        compiler_params=pltpu.CompilerParams(
            needs_layout_passes=False, use_tc_tiling_on_sc=False),
    )
    out = sc(ex_lin, wt8.reshape(-1), qt8.reshape(-1), cms3.reshape(-1),
             tiled_view(exercise_id), tiled_view(weight_id),
             tiled_view(equipment_id), tiled_view(exercise_sequence),
             tiled_view(core), tiled_view(metric_type))
    return (out.reshape(L, n_bb, D_OUT, TILE_B).transpose(1, 3, 0, 2)
            .reshape(B, L, D_OUT))


# speedup vs baseline: 1.0876x; 1.0876x over previous
import functools

import jax
import jax.numpy as jnp
from jax import lax
from jax.experimental import pallas as pl
from jax.experimental.pallas import tpu as pltpu
from jax.experimental.pallas import tpu_sc as plsc

D_OUT = 8
NC, NS = 2, 16
NW = NC * NS
TILE_L, TILE_B = 8, 128
SLAB = TILE_L * TILE_B
EX_CHK = 4000


def _prep_body(wt_ref, st_ref, qt_ref, ct_ref, mt_ref,
               ww_ref, sw_ref, qw_ref, cw_ref, mw_ref,
               wb_ref, sb_ref, qb_ref, cb_ref, mb_ref,
               wt8_ref, qt8_ref, cms_ref):
    f32 = jnp.float32
    wt8_ref[...] = jnp.dot(wt_ref[...], ww_ref[...].T, preferred_element_type=f32)
    qt8_ref[...] = jnp.dot(qt_ref[...], qw_ref[...][:, :2].T, preferred_element_type=f32)
    st8 = jnp.dot(st_ref[...], sw_ref[...].T, preferred_element_type=f32)
    ct8 = jnp.dot(ct_ref[...], cw_ref[...].T, preferred_element_type=f32)
    mt8 = jnp.dot(mt_ref[...], mw_ref[...].T, preferred_element_type=f32)
    bias = wb_ref[...] + sb_ref[...] + qb_ref[...] + cb_ref[...] + mb_ref[...]
    cm = (ct8[:, None, :] + mt8[None, :, :]).reshape(8, D_OUT)
    cms = st8[:, None, :] + cm[None, :, :] + bias[None, None, :]
    cms_ref[...] = cms


def _transpose_body(src_ref, dst_ref):
    dst_ref[...] = src_ref[...].T


def _sc_body(ex_t, wt8, qt8, cms, g_ex, g_w, g_q, g_s, g_c, g_m, out,
             idx, rows, acc, twt, tqt, tcms, stage, sem_i, sem_g, sem_o, sem_t,
             *, n_lb):
    w = lax.axis_index("s") * NC + lax.axis_index("c")
    lane = lax.iota(jnp.int32, 16)

    p9 = (lane >> 3) * 9 + (lane & 7)
    for src, dst in ((wt8, twt), (qt8, tqt), (cms, tcms)):
        cp = pltpu.make_async_copy(src, stage.at[pl.ds(0, src.shape[0])], sem_t)
        cp.start()
        cp.wait()

        @plsc.parallel_loop(0, src.shape[0] // 16, unroll=4)
        def _expand(k, dst=dst):
            plsc.store_scatter(dst, [p9 + k * 18], stage[pl.ds(k * 16, 16)])

    def fire_idx(lb, s):
        gi = idx[s]
        for i, g in enumerate((g_ex, g_w, g_q, g_s, g_c, g_m)):
            pltpu.make_async_copy(g.at[lb, w], gi[i], sem_i[s]).start()

    def finish_idx_fold_gather(lb, s):
        gi = idx[s]
        with jax.named_scope("idx_wait"):
            for i, g in enumerate((g_ex, g_w, g_q, g_s, g_c, g_m)):
                pltpu.make_async_copy(g.at[lb, w], gi[i], sem_i[s]).wait()

        @plsc.parallel_loop(0, SLAB // 16, unroll=4)
        def _fold(k):
            ks = pl.ds(k * 16, 16)
            gi[1][ks] = gi[1][ks] * 9
            gi[2][ks] = gi[2][ks] * 9
            gi[3][ks] = gi[3][ks] * 72 + gi[4][ks] * 36 + gi[5][ks] * 9

        pltpu.make_async_copy(ex_t.at[gi[0]], rows[s], sem_g[s]).start()

    def load_fold_gather(lb, s):
        fire_idx(lb, s)
        finish_idx_fold_gather(lb, s)

    def do_round(r, s):
        @pl.when(r + 1 < n_lb)
        def _prefetch_fire():
            fire_idx(r + 1, 1 - s)

        with jax.named_scope("gather_wait"):
            pltpu.make_async_copy(ex_t.at[idx[s][0]], rows[s], sem_g[s]).wait()

        @pl.when(r >= 2)
        def _drain_out():
            pltpu.make_async_copy(acc[s], out.at[pl.ds((r - 2) * TILE_L, TILE_L), w],
                                  sem_o[s]).wait()

        gi, rex, a = idx[s], rows[s], acc[s]
        iw, iq, icms = gi[1], gi[2], gi[3]

        zero16 = jnp.broadcast_to(jnp.int32(0), (16,))

        sc_add = jax.named_scope("adds")
        sc_add.__enter__()

        @plsc.parallel_loop(0, SLAB // 16, unroll=2)
        def _add(j):
            js = pl.ds(j * 16, 16)
            tok = j * 16 + lane
            w8 = iw[js]
            q8 = iq[js]
            c8 = icms[js]
            li = j >> 3
            base = (j & 7) * 16
            for d in range(D_OUT):
                v = (plsc.load_gather(rex, [tok, zero16 + d])
                     + plsc.load_gather(twt, [w8])
                     + plsc.load_gather(tqt, [q8])
                     + plsc.load_gather(tcms, [c8]))
                a[li, pl.ds(d * TILE_B + base, 16)] = v
                if d < D_OUT - 1:
                    w8 = w8 + 1
                    q8 = q8 + 1
                    c8 = c8 + 1

        sc_add.__exit__(None, None, None)

        @pl.when(r + 1 < n_lb)
        def _prefetch_finish():
            finish_idx_fold_gather(r + 1, 1 - s)

        pltpu.make_async_copy(a, out.at[pl.ds(r * TILE_L, TILE_L), w],
                              sem_o[s]).start()

    load_fold_gather(0, 0)

    @pl.loop(0, (n_lb + 1) // 2)
    def _round_pair(h):
        for sub in (0, 1):
            r = h * 2 + sub

            @pl.when(r < n_lb)
            def _do(r=r, sub=sub):
                do_round(r, sub)

    for s, r in ((n_lb % 2, n_lb - 2), ((n_lb - 1) % 2, n_lb - 1)):
        pltpu.make_async_copy(acc[s], out.at[pl.ds(r * TILE_L, TILE_L), w],
                              sem_o[s]).wait()


def kernel(exercise_id, weight_id, exercise_sequence, equipment_id, core, metric_type,
           exercise_table, weight_table, seq_table, equipment_table, core_table, metric_table,
           weight_fc_w, weight_fc_b, seq_fc_w, seq_fc_b, equipment_fc_w, equipment_fc_b,
           core_fc_w, core_fc_b, metric_fc_w, metric_fc_b):
    B, L = exercise_id.shape
    V = exercise_table.shape[0]
    f32 = jnp.float32
    n_lb, n_bb = L // TILE_L, B // TILE_B

    wt8, qt8, cms3 = pl.pallas_call(
        _prep_body,
        out_shape=[
            jax.ShapeDtypeStruct(weight_table.shape[:1] + (D_OUT,), f32),
            jax.ShapeDtypeStruct(equipment_table.shape[:1] + (D_OUT,), f32),
            jax.ShapeDtypeStruct((seq_table.shape[0], 8, D_OUT), f32),
        ],
    )(weight_table, seq_table, equipment_table, core_table, metric_table,
      weight_fc_w, seq_fc_w, equipment_fc_w, core_fc_w, metric_fc_w,
      weight_fc_b, seq_fc_b, equipment_fc_b, core_fc_b, metric_fc_b)

    ex_lin = pl.pallas_call(
        _transpose_body,
        out_shape=jax.ShapeDtypeStruct((V, D_OUT), f32),
    )(exercise_table.T)

    def tiled_view(a):
        return (a.T.reshape(n_lb, TILE_L, n_bb, TILE_B)
                .transpose(0, 2, 1, 3).reshape(n_lb, n_bb, SLAB))

    mesh = plsc.VectorSubcoreMesh(core_axis_name="c", subcore_axis_name="s",
                                  num_cores=NC, num_subcores=NS)
    sc = pl.kernel(
        functools.partial(_sc_body, n_lb=n_lb),
        out_type=jax.ShapeDtypeStruct((L, n_bb, SLAB), f32),
        mesh=mesh,
        scratch_types=[
            [[pltpu.VMEM((SLAB,), jnp.int32) for _ in range(6)] for _ in range(2)],
            [pltpu.VMEM((SLAB, D_OUT), f32) for _ in range(2)],
            [pltpu.VMEM((TILE_L, SLAB), f32) for _ in range(2)],
            pltpu.VMEM((weight_table.shape[0] * 9,), f32),
            pltpu.VMEM((equipment_table.shape[0] * 9,), f32),
            pltpu.VMEM((seq_table.shape[0] * 8 * 9,), f32),
            pltpu.VMEM((seq_table.shape[0] * 8 * D_OUT,), f32),
            [pltpu.SemaphoreType.DMA for _ in range(2)],
            [pltpu.SemaphoreType.DMA for _ in range(2)],
            [pltpu.SemaphoreType.DMA for _ in range(2)],
            pltpu.SemaphoreType.DMA,
        ],
        compiler_params=pltpu.CompilerParams(
            needs_layout_passes=False, use_tc_tiling_on_sc=False),
    )
    out = sc(ex_lin, wt8.reshape(-1), qt8.reshape(-1), cms3.reshape(-1),
             tiled_view(exercise_id), tiled_view(weight_id),
             tiled_view(equipment_id), tiled_view(exercise_sequence),
             tiled_view(core), tiled_view(metric_type))
    return (out.reshape(L, n_bb, D_OUT, TILE_B).transpose(1, 3, 0, 2)
            .reshape(B, L, D_OUT))


# speedup vs baseline: 1.2077x; 1.1104x over previous
import functools

import jax
import jax.numpy as jnp
from jax import lax
from jax.experimental import pallas as pl
from jax.experimental.pallas import tpu as pltpu
from jax.experimental.pallas import tpu_sc as plsc

D_OUT = 8
NC, NS = 2, 16
NW = NC * NS
TILE_L, TILE_B = 8, 128
SLAB = TILE_L * TILE_B
EX_CHK = 4000


def _prep_body(wt_ref, st_ref, qt_ref, ct_ref, mt_ref,
               ww_ref, sw_ref, qw_ref, cw_ref, mw_ref,
               wb_ref, sb_ref, qb_ref, cb_ref, mb_ref,
               wt8_ref, qt8_ref, cms_ref):
    f32 = jnp.float32
    wt8_ref[...] = jnp.dot(wt_ref[...], ww_ref[...].T, preferred_element_type=f32)
    qt8_ref[...] = jnp.dot(qt_ref[...], qw_ref[...][:, :2].T, preferred_element_type=f32)
    st8 = jnp.dot(st_ref[...], sw_ref[...].T, preferred_element_type=f32)
    ct8 = jnp.dot(ct_ref[...], cw_ref[...].T, preferred_element_type=f32)
    mt8 = jnp.dot(mt_ref[...], mw_ref[...].T, preferred_element_type=f32)
    bias = wb_ref[...] + sb_ref[...] + qb_ref[...] + cb_ref[...] + mb_ref[...]
    cm = (ct8[:, None, :] + mt8[None, :, :]).reshape(8, D_OUT)
    cms = st8[:, None, :] + cm[None, :, :] + bias[None, None, :]
    cms_ref[...] = cms


def _transpose_body(src_ref, dst_ref):
    dst_ref[...] = src_ref[...].T


def _sc_body(ex_t, wt8, qt8, cms, g_ex, g_w, g_q, g_s, g_c, g_m, out,
             idx, rows, acc, twt, tqt, tcms, stage, sem_i, sem_g, sem_o, sem_t,
             *, n_lb):
    w = lax.axis_index("s") * NC + lax.axis_index("c")
    lane = lax.iota(jnp.int32, 16)

    p9 = (lane >> 3) * 9 + (lane & 7)
    for src, dst in ((wt8, twt), (qt8, tqt), (cms, tcms)):
        cp = pltpu.make_async_copy(src, stage.at[pl.ds(0, src.shape[0])], sem_t)
        cp.start()
        cp.wait()

        @plsc.parallel_loop(0, src.shape[0] // 16, unroll=4)
        def _expand(k, dst=dst):
            plsc.store_scatter(dst, [p9 + k * 18], stage[pl.ds(k * 16, 16)])

    def fire_idx(lb, s):
        gi = idx[s]
        for i, g in enumerate((g_ex, g_w, g_q, g_s, g_c, g_m)):
            pltpu.make_async_copy(g.at[lb, w], gi[i], sem_i[s]).start()

    def finish_idx_fold_gather(lb, s):
        gi = idx[s]
        with jax.named_scope("idx_wait"):
            for i, g in enumerate((g_ex, g_w, g_q, g_s, g_c, g_m)):
                pltpu.make_async_copy(g.at[lb, w], gi[i], sem_i[s]).wait()

        @plsc.parallel_loop(0, SLAB // 16, unroll=4)
        def _fold(k):
            ks = pl.ds(k * 16, 16)
            gi[1][ks] = gi[1][ks] * 9
            gi[2][ks] = gi[2][ks] * 9
            gi[3][ks] = gi[3][ks] * 72 + gi[4][ks] * 36 + gi[5][ks] * 9

        pltpu.make_async_copy(ex_t.at[gi[0]], rows[s], sem_g[s]).start()

    def load_fold_gather(lb, s):
        fire_idx(lb, s)
        finish_idx_fold_gather(lb, s)

    def do_round(r, s):
        @pl.when(r + 1 < n_lb)
        def _prefetch():
            load_fold_gather(r + 1, 1 - s)

        with jax.named_scope("gather_wait"):
            pltpu.make_async_copy(ex_t.at[idx[s][0]], rows[s], sem_g[s]).wait()

        @pl.when(r >= 2)
        def _drain_out():
            pltpu.make_async_copy(acc[s], out.at[pl.ds((r - 2) * TILE_L, TILE_L), w],
                                  sem_o[s]).wait()

        gi, rex, a = idx[s], rows[s], acc[s]
        iw, iq, icms = gi[1], gi[2], gi[3]

        zero16 = jnp.broadcast_to(jnp.int32(0), (16,))

        sc_add = jax.named_scope("adds")
        sc_add.__enter__()

        @plsc.parallel_loop(0, SLAB // 16, unroll=2)
        def _add(j):
            js = pl.ds(j * 16, 16)
            tok = j * 16 + lane
            w8 = iw[js]
            q8 = iq[js]
            c8 = icms[js]
            li = j >> 3
            base = (j & 7) * 16
            for d in range(D_OUT):
                v = (plsc.load_gather(rex, [tok, zero16 + d])
                     + plsc.load_gather(twt, [w8])
                     + plsc.load_gather(tqt, [q8])
                     + plsc.load_gather(tcms, [c8]))
                a[li, pl.ds(d * TILE_B + base, 16)] = v
                if d < D_OUT - 1:
                    w8 = w8 + 1
                    q8 = q8 + 1
                    c8 = c8 + 1

        sc_add.__exit__(None, None, None)
        pltpu.make_async_copy(a, out.at[pl.ds(r * TILE_L, TILE_L), w],
                              sem_o[s]).start()

    load_fold_gather(0, 0)

    @pl.loop(0, (n_lb + 1) // 2)
    def _round_pair(h):
        for sub in (0, 1):
            r = h * 2 + sub

            @pl.when(r < n_lb)
            def _do(r=r, sub=sub):
                do_round(r, sub)

    for s, r in ((n_lb % 2, n_lb - 2), ((n_lb - 1) % 2, n_lb - 1)):
        pltpu.make_async_copy(acc[s], out.at[pl.ds(r * TILE_L, TILE_L), w],
                              sem_o[s]).wait()


def kernel(exercise_id, weight_id, exercise_sequence, equipment_id, core, metric_type,
           exercise_table, weight_table, seq_table, equipment_table, core_table, metric_table,
           weight_fc_w, weight_fc_b, seq_fc_w, seq_fc_b, equipment_fc_w, equipment_fc_b,
           core_fc_w, core_fc_b, metric_fc_w, metric_fc_b):
    B, L = exercise_id.shape
    V = exercise_table.shape[0]
    f32 = jnp.float32
    n_lb, n_bb = L // TILE_L, B // TILE_B

    wt8, qt8, cms3 = pl.pallas_call(
        _prep_body,
        out_shape=[
            jax.ShapeDtypeStruct(weight_table.shape[:1] + (D_OUT,), f32),
            jax.ShapeDtypeStruct(equipment_table.shape[:1] + (D_OUT,), f32),
            jax.ShapeDtypeStruct((seq_table.shape[0], 8, D_OUT), f32),
        ],
    )(weight_table, seq_table, equipment_table, core_table, metric_table,
      weight_fc_w, seq_fc_w, equipment_fc_w, core_fc_w, metric_fc_w,
      weight_fc_b, seq_fc_b, equipment_fc_b, core_fc_b, metric_fc_b)

    ex_lin = pl.pallas_call(
        _transpose_body,
        out_shape=jax.ShapeDtypeStruct((V, D_OUT), f32),
    )(exercise_table.T)

    def tiled_view(a):
        return (a.T.reshape(n_lb, TILE_L, n_bb, TILE_B)
                .transpose(0, 2, 1, 3).reshape(n_lb, n_bb, SLAB))

    mesh = plsc.VectorSubcoreMesh(core_axis_name="c", subcore_axis_name="s",
                                  num_cores=NC, num_subcores=NS)
    sc = pl.kernel(
        functools.partial(_sc_body, n_lb=n_lb),
        out_type=jax.ShapeDtypeStruct((L, n_bb, SLAB), f32),
        mesh=mesh,
        scratch_types=[
            [[pltpu.VMEM((SLAB,), jnp.int32) for _ in range(6)] for _ in range(2)],
            [pltpu.VMEM((SLAB, D_OUT), f32) for _ in range(2)],
            [pltpu.VMEM((TILE_L, SLAB), f32) for _ in range(2)],
            pltpu.VMEM((weight_table.shape[0] * 9,), f32),
            pltpu.VMEM((equipment_table.shape[0] * 9,), f32),
            pltpu.VMEM((seq_table.shape[0] * 8 * 9,), f32),
            pltpu.VMEM((seq_table.shape[0] * 8 * D_OUT,), f32),
            [pltpu.SemaphoreType.DMA for _ in range(2)],
            [pltpu.SemaphoreType.DMA for _ in range(2)],
            [pltpu.SemaphoreType.DMA for _ in range(2)],
            pltpu.SemaphoreType.DMA,
        ],
        compiler_params=pltpu.CompilerParams(
            needs_layout_passes=False, use_tc_tiling_on_sc=False),
    )
    out = sc(ex_lin, wt8.reshape(-1), qt8.reshape(-1), cms3.reshape(-1),
             tiled_view(exercise_id), tiled_view(weight_id),
             tiled_view(equipment_id), tiled_view(exercise_sequence),
             tiled_view(core), tiled_view(metric_type))
    return (out.reshape(L, n_bb, D_OUT, TILE_B).transpose(1, 3, 0, 2)
            .reshape(B, L, D_OUT))


# speedup vs baseline: 1.3024x; 1.0784x over previous
import functools

import jax
import jax.numpy as jnp
from jax import lax
from jax.experimental import pallas as pl
from jax.experimental.pallas import tpu as pltpu
from jax.experimental.pallas import tpu_sc as plsc

D_OUT = 8
NC, NS = 2, 16
NW = NC * NS
TILE_L, TILE_B = 8, 128
SLAB = TILE_L * TILE_B
EX_CHK = 4000


def _prep_body(wt_ref, st_ref, qt_ref, ct_ref, mt_ref,
               ww_ref, sw_ref, qw_ref, cw_ref, mw_ref,
               wb_ref, sb_ref, qb_ref, cb_ref, mb_ref,
               wt8_ref, qt8_ref, cms_ref):
    f32 = jnp.float32
    wt8_ref[...] = jnp.dot(wt_ref[...], ww_ref[...].T, preferred_element_type=f32)
    qt8_ref[...] = jnp.dot(qt_ref[...], qw_ref[...][:, :2].T, preferred_element_type=f32)
    st8 = jnp.dot(st_ref[...], sw_ref[...].T, preferred_element_type=f32)
    ct8 = jnp.dot(ct_ref[...], cw_ref[...].T, preferred_element_type=f32)
    mt8 = jnp.dot(mt_ref[...], mw_ref[...].T, preferred_element_type=f32)
    bias = wb_ref[...] + sb_ref[...] + qb_ref[...] + cb_ref[...] + mb_ref[...]
    cm = (ct8[:, None, :] + mt8[None, :, :]).reshape(8, D_OUT)
    cms = st8[:, None, :] + cm[None, :, :] + bias[None, None, :]
    cms_ref[...] = cms


def _transpose_body(src_ref, dst_ref):
    dst_ref[...] = src_ref[...].T


def _sc_body(ex_t, wt8, qt8, cms, g_ex, g_w, g_q, g_s, g_c, g_m, out,
             idx, rows, acc, twt, tqt, tcms, stage, sem_i, sem_g, sem_o, sem_t,
             *, n_lb):
    w = lax.axis_index("s") * NC + lax.axis_index("c")
    lane = lax.iota(jnp.int32, 16)

    p9 = (lane >> 3) * 9 + (lane & 7)
    for src, dst in ((wt8, twt), (qt8, tqt), (cms, tcms)):
        cp = pltpu.make_async_copy(src, stage.at[pl.ds(0, src.shape[0])], sem_t)
        cp.start()
        cp.wait()

        @plsc.parallel_loop(0, src.shape[0] // 16, unroll=4)
        def _expand(k, dst=dst):
            plsc.store_scatter(dst, [p9 + k * 18], stage[pl.ds(k * 16, 16)])

    def fire_idx(lb, s):
        gi = idx[s]
        for i, g in enumerate((g_ex, g_w, g_q, g_s, g_c, g_m)):
            pltpu.make_async_copy(g.at[lb, w], gi[i], sem_i[s]).start()

    def finish_idx_fold_gather(lb, s4, s):
        gi = idx[s4]
        with jax.named_scope("idx_wait"):
            for i, g in enumerate((g_ex, g_w, g_q, g_s, g_c, g_m)):
                pltpu.make_async_copy(g.at[lb, w], gi[i], sem_i[s4]).wait()

        @plsc.parallel_loop(0, SLAB // 16, unroll=4)
        def _fold(k):
            ks = pl.ds(k * 16, 16)
            gi[1][ks] = gi[1][ks] * 9
            gi[2][ks] = gi[2][ks] * 9
            gi[3][ks] = gi[3][ks] * 72 + gi[4][ks] * 36 + gi[5][ks] * 9

        pltpu.make_async_copy(ex_t.at[gi[0]], rows[s], sem_g[s]).start()

    def do_round(r, s, s4):
        @pl.when(r + 2 < n_lb)
        def _fire_ahead():
            fire_idx(r + 2, (s4 + 2) % 4)

        @pl.when(r + 1 < n_lb)
        def _prefetch():
            finish_idx_fold_gather(r + 1, (s4 + 1) % 4, 1 - s)

        with jax.named_scope("gather_wait"):
            pltpu.make_async_copy(ex_t.at[idx[s4][0]], rows[s], sem_g[s]).wait()

        @pl.when(r >= 2)
        def _drain_out():
            pltpu.make_async_copy(acc[s], out.at[pl.ds((r - 2) * TILE_L, TILE_L), w],
                                  sem_o[s]).wait()

        gi, rex, a = idx[s4], rows[s], acc[s]
        iw, iq, icms = gi[1], gi[2], gi[3]

        zero16 = jnp.broadcast_to(jnp.int32(0), (16,))

        sc_add = jax.named_scope("adds")
        sc_add.__enter__()

        @plsc.parallel_loop(0, SLAB // 16, unroll=2)
        def _add(j):
            js = pl.ds(j * 16, 16)
            tok = j * 16 + lane
            w8 = iw[js]
            q8 = iq[js]
            c8 = icms[js]
            li = j >> 3
            base = (j & 7) * 16
            for d in range(D_OUT):
                v = (plsc.load_gather(rex, [tok, zero16 + d])
                     + plsc.load_gather(twt, [w8])
                     + plsc.load_gather(tqt, [q8])
                     + plsc.load_gather(tcms, [c8]))
                a[li, pl.ds(d * TILE_B + base, 16)] = v
                if d < D_OUT - 1:
                    w8 = w8 + 1
                    q8 = q8 + 1
                    c8 = c8 + 1

        sc_add.__exit__(None, None, None)
        pltpu.make_async_copy(a, out.at[pl.ds(r * TILE_L, TILE_L), w],
                              sem_o[s]).start()

    fire_idx(0, 0)
    finish_idx_fold_gather(0, 0, 0)
    fire_idx(1, 1)

    @pl.loop(0, (n_lb + 3) // 4)
    def _round_quad(h):
        for sub in (0, 1, 2, 3):
            r = h * 4 + sub

            @pl.when(r < n_lb)
            def _do(r=r, sub=sub):
                do_round(r, sub % 2, sub)

    for s, r in ((n_lb % 2, n_lb - 2), ((n_lb - 1) % 2, n_lb - 1)):
        pltpu.make_async_copy(acc[s], out.at[pl.ds(r * TILE_L, TILE_L), w],
                              sem_o[s]).wait()


def kernel(exercise_id, weight_id, exercise_sequence, equipment_id, core, metric_type,
           exercise_table, weight_table, seq_table, equipment_table, core_table, metric_table,
           weight_fc_w, weight_fc_b, seq_fc_w, seq_fc_b, equipment_fc_w, equipment_fc_b,
           core_fc_w, core_fc_b, metric_fc_w, metric_fc_b):
    B, L = exercise_id.shape
    V = exercise_table.shape[0]
    f32 = jnp.float32
    n_lb, n_bb = L // TILE_L, B // TILE_B

    wt8, qt8, cms3 = pl.pallas_call(
        _prep_body,
        out_shape=[
            jax.ShapeDtypeStruct(weight_table.shape[:1] + (D_OUT,), f32),
            jax.ShapeDtypeStruct(equipment_table.shape[:1] + (D_OUT,), f32),
            jax.ShapeDtypeStruct((seq_table.shape[0], 8, D_OUT), f32),
        ],
    )(weight_table, seq_table, equipment_table, core_table, metric_table,
      weight_fc_w, seq_fc_w, equipment_fc_w, core_fc_w, metric_fc_w,
      weight_fc_b, seq_fc_b, equipment_fc_b, core_fc_b, metric_fc_b)

    ex_lin = pl.pallas_call(
        _transpose_body,
        out_shape=jax.ShapeDtypeStruct((V, D_OUT), f32),
    )(exercise_table.T)

    def tiled_view(a):
        return (a.T.reshape(n_lb, TILE_L, n_bb, TILE_B)
                .transpose(0, 2, 1, 3).reshape(n_lb, n_bb, SLAB))

    mesh = plsc.VectorSubcoreMesh(core_axis_name="c", subcore_axis_name="s",
                                  num_cores=NC, num_subcores=NS)
    sc = pl.kernel(
        functools.partial(_sc_body, n_lb=n_lb),
        out_type=jax.ShapeDtypeStruct((L, n_bb, SLAB), f32),
        mesh=mesh,
        scratch_types=[
            [[pltpu.VMEM((SLAB,), jnp.int32) for _ in range(6)] for _ in range(4)],
            [pltpu.VMEM((SLAB, D_OUT), f32) for _ in range(2)],
            [pltpu.VMEM((TILE_L, SLAB), f32) for _ in range(2)],
            pltpu.VMEM((weight_table.shape[0] * 9,), f32),
            pltpu.VMEM((equipment_table.shape[0] * 9,), f32),
            pltpu.VMEM((seq_table.shape[0] * 8 * 9,), f32),
            pltpu.VMEM((seq_table.shape[0] * 8 * D_OUT,), f32),
            [pltpu.SemaphoreType.DMA for _ in range(4)],
            [pltpu.SemaphoreType.DMA for _ in range(2)],
            [pltpu.SemaphoreType.DMA for _ in range(2)],
            pltpu.SemaphoreType.DMA,
        ],
        compiler_params=pltpu.CompilerParams(
            needs_layout_passes=False, use_tc_tiling_on_sc=False),
    )
    out = sc(ex_lin, wt8.reshape(-1), qt8.reshape(-1), cms3.reshape(-1),
             tiled_view(exercise_id), tiled_view(weight_id),
             tiled_view(equipment_id), tiled_view(exercise_sequence),
             tiled_view(core), tiled_view(metric_type))
    return (out.reshape(L, n_bb, D_OUT, TILE_B).transpose(1, 3, 0, 2)
            .reshape(B, L, D_OUT))
